# Initial kernel scaffold; baseline (speedup 1.0000x reference)
import jax
import jax.numpy as jnp
from jax import lax
from jax.experimental import pallas as pl
from jax.experimental.pallas import tpu as pltpu
from jax.experimental.pallas import tpu_sc as plsc

_OUT = 7
_NK = 4
_NC = 2
_NS = 16
_NW = _NC * _NS
_LANES = 16


def _pyramid_body(ft_ref, out_ref, xt_ref, yt_ref):
    t = pl.program_id(0)
    kx = t // _NK
    ky = t % _NK

    @pl.when(t == 0)
    def _():
        xt_ref[...] = ft_ref[...]

    for k in range(1, _NK):
        s = 1 << (k - 1)

        @pl.when((ky == 0) & (kx == k))
        def _():
            xt = xt_ref[...]
            shifted = jnp.concatenate([xt[:, s:, :]] + [xt[:, -1:, :]] * s, axis=1)
            xt_ref[...] = jnp.maximum(xt, shifted)

    @pl.when(ky == 0)
    def _():
        yt_ref[...] = xt_ref[...]

    for k in range(1, _NK):
        s = 1 << (k - 1)

        @pl.when(ky == k)
        def _():
            yt = yt_ref[...]
            shifted = jnp.concatenate([yt[s:]] + [yt[-1:]] * s, axis=0)
            yt_ref[...] = jnp.maximum(yt, shifted)

    out_ref[0] = yt_ref[...]


def _build_pyramid(ft, interpret=False):
    h, w, c = ft.shape
    return pl.pallas_call(
        _pyramid_body,
        grid=(_NK * _NK,),
        in_specs=[pl.BlockSpec((h, w, c), lambda t: (0, 0, 0))],
        out_specs=pl.BlockSpec((1, h, w, c), lambda t: (t, 0, 0, 0)),
        out_shape=jax.ShapeDtypeStruct((_NK * _NK, h, w, c), jnp.float32),
        scratch_shapes=[
            pltpu.VMEM((h, w, c), jnp.float32),
            pltpu.VMEM((h, w, c), jnp.float32),
        ],
        interpret=interpret,
    )(ft)


def _make_sc_pool(h, w, c, n_rois, rpw, interpret=False):
    nbins = _OUT * _OUT
    plane = h * w

    def body(pyr_hbm, rois_hbm, out_hbm, rois_v, yp_v, buf_v, acc_v, sem):
        cid = lax.axis_index("c")
        sid = lax.axis_index("s")
        wid = cid * _NS + sid
        start = wid * rpw

        pltpu.sync_copy(rois_hbm.at[pl.ds(start, rpw)], rois_v)

        lane = lax.iota(jnp.int32, _LANES)
        l8 = lane & 7
        hi_half = lane >= 8
        zero16 = jnp.zeros((_LANES,), jnp.int32)
        one16 = zero16 + 1

        def do_roi(r, carry):
            rv = zero16 + r
            x1 = plsc.load_gather(rois_v, [rv, zero16])
            y1 = plsc.load_gather(rois_v, [rv, one16])
            x2 = plsc.load_gather(rois_v, [rv, zero16 + 2])
            y2 = plsc.load_gather(rois_v, [rv, zero16 + 3])
            rw = x2 - x1 + 1
            rh = y2 - y1 + 1

            xb = x1 + (rw * l8) // _OUT
            xe = x1 + (rw * (l8 + 1)) // _OUT
            lx = xe - xb
            kxv = ((lx >= 2).astype(jnp.int32) + (lx >= 4).astype(jnp.int32)
                   + (lx >= 8).astype(jnp.int32))
            xhi = xe - (one16 << kxv)
            xpt = jnp.where(hi_half, xhi, xb)
            xbase = kxv * (_NK * plane) + xpt

            yb = y1 + (rh * l8) // _OUT
            ye = y1 + (rh * (l8 + 1)) // _OUT
            ly = ye - yb
            kyv = ((ly >= 2).astype(jnp.int32) + (ly >= 4).astype(jnp.int32)
                   + (ly >= 8).astype(jnp.int32))
            yhi = ye - (one16 << kyv)
            yp_v[0] = kyv * plane + yb * w
            yp_v[1] = kyv * plane + yhi * w

            copies = []
            for by in range(_OUT):
                bv = zero16 + by
                for sy in range(2):
                    yterm = plsc.load_gather(yp_v, [zero16 + sy, bv])
                    idx = xbase + yterm
                    row = (by * 2 + sy) * _LANES
                    copies.append(pltpu.async_copy(
                        pyr_hbm.at[idx], buf_v.at[pl.ds(row, _LANES)], sem))
            for cp in copies:
                cp.wait()

            def cbody(cc, _):
                cs = cc * _LANES
                for by in range(_OUT):
                    rbase = (by * 2) * _LANES
                    for bx in range(_OUT):
                        v00 = buf_v[rbase + bx, pl.ds(cs, _LANES)]
                        v01 = buf_v[rbase + bx + 8, pl.ds(cs, _LANES)]
                        v10 = buf_v[rbase + _LANES + bx, pl.ds(cs, _LANES)]
                        v11 = buf_v[rbase + _LANES + bx + 8, pl.ds(cs, _LANES)]
                        acc_v[by * _OUT + bx, pl.ds(cs, _LANES)] = jnp.maximum(
                            jnp.maximum(v00, v01), jnp.maximum(v10, v11))
                return 0

            lax.fori_loop(0, c // _LANES, cbody, 0)

            gr = start + r

            @pl.when(gr < n_rois)
            def _():
                pltpu.sync_copy(acc_v, out_hbm.at[pl.ds(gr * nbins, nbins)])

            return carry

        lax.fori_loop(0, rpw, do_roi, 0)

    mesh = plsc.VectorSubcoreMesh(core_axis_name="c", subcore_axis_name="s")
    return pl.kernel(
        body,
        out_type=jax.ShapeDtypeStruct((n_rois * nbins, c), jnp.float32),
        mesh=mesh,
        scratch_types=[
            pltpu.VMEM((rpw, 8), jnp.int32),
            pltpu.VMEM((2, _LANES), jnp.int32),
            pltpu.VMEM((2 * _OUT * _LANES, c), jnp.float32),
            pltpu.VMEM((nbins, c), jnp.float32),
            pltpu.SemaphoreType.DMA,
        ],
        interpret=interpret,
    )


def kernel(features, rois):
    _, c, h, w = features.shape
    n_rois = rois.shape[0]
    rpw = -(-n_rois // _NW)

    ft = jnp.transpose(features[0], (1, 2, 0))
    pyr = _build_pyramid(ft)
    pyr_rows = pyr.reshape(_NK * _NK * h * w, c)

    pad = jnp.tile(jnp.array([0, 0, 6, 6, 0, 0, 0, 0], jnp.int32), (rpw * _NW, 1))
    rois_p = pad.at[:n_rois, :4].set(rois.astype(jnp.int32))

    out_flat = _make_sc_pool(h, w, c, n_rois, rpw)(pyr_rows, rois_p)
    return out_flat.reshape(n_rois, _OUT, _OUT, c).transpose(0, 3, 1, 2)


# speedup vs baseline: 32.5679x; 32.5679x over previous
import jax
import jax.numpy as jnp
from jax import lax
from jax.experimental import pallas as pl
from jax.experimental.pallas import tpu as pltpu
from jax.experimental.pallas import tpu_sc as plsc

_OUT = 7
_NK = 4
_NC = 2
_NS = 16
_NW = _NC * _NS
_LANES = 16


def _pyramid_body(ft_ref, out_ref, xt_ref, yt_ref):
    t = pl.program_id(0)
    kx = t // _NK
    ky = t % _NK

    @pl.when(t == 0)
    def _():
        xt_ref[...] = ft_ref[...]

    for k in range(1, _NK):
        s = 1 << (k - 1)

        @pl.when((ky == 0) & (kx == k))
        def _():
            xt = xt_ref[...]
            shifted = jnp.concatenate([xt[:, s:, :]] + [xt[:, -1:, :]] * s, axis=1)
            xt_ref[...] = jnp.maximum(xt, shifted)

    @pl.when(ky == 0)
    def _():
        yt_ref[...] = xt_ref[...]

    for k in range(1, _NK):
        s = 1 << (k - 1)

        @pl.when(ky == k)
        def _():
            yt = yt_ref[...]
            shifted = jnp.concatenate([yt[s:]] + [yt[-1:]] * s, axis=0)
            yt_ref[...] = jnp.maximum(yt, shifted)

    out_ref[0] = yt_ref[...]


def _build_pyramid(ft, interpret=False):
    h, w, c = ft.shape
    return pl.pallas_call(
        _pyramid_body,
        grid=(_NK * _NK,),
        in_specs=[pl.BlockSpec((h, w, c), lambda t: (0, 0, 0))],
        out_specs=pl.BlockSpec((1, h, w, c), lambda t: (t, 0, 0, 0)),
        out_shape=jax.ShapeDtypeStruct((_NK * _NK, h, w, c), jnp.float32),
        scratch_shapes=[
            pltpu.VMEM((h, w, c), jnp.float32),
            pltpu.VMEM((h, w, c), jnp.float32),
        ],
        interpret=interpret,
    )(ft)


_OROWS = 56


def _make_sc_pool(h, w, c, n_rois, rpw, interpret=False):
    plane = h * w

    def body(pyr_hbm, rois_hbm, out_hbm, rois_v, yp_v, buf_v, acc_v, sem):
        cid = lax.axis_index("c")
        sid = lax.axis_index("s")
        wid = cid * _NS + sid
        start = wid * rpw

        pltpu.sync_copy(rois_hbm.at[pl.ds(start * 8, rpw * 8)], rois_v)

        lane = lax.iota(jnp.int32, _LANES)
        l8 = lane & 7
        hi_half = lane >= 8
        zero16 = jnp.zeros((_LANES,), jnp.int32)
        one16 = zero16 + 1

        def do_roi(r, carry):
            rv = zero16 + r * 8
            x1 = plsc.load_gather(rois_v, [rv])
            y1 = plsc.load_gather(rois_v, [rv + 1])
            x2 = plsc.load_gather(rois_v, [rv + 2])
            y2 = plsc.load_gather(rois_v, [rv + 3])
            rw = x2 - x1 + 1
            rh = y2 - y1 + 1

            xb = x1 + (rw * l8) // _OUT
            xe = x1 + (rw * (l8 + 1)) // _OUT
            lx = xe - xb
            kxv = ((lx >= 2).astype(jnp.int32) + (lx >= 4).astype(jnp.int32)
                   + (lx >= 8).astype(jnp.int32))
            xhi = xe - (one16 << kxv)
            xpt = jnp.where(hi_half, xhi, xb)
            xbase = kxv * (_NK * plane) + xpt

            yb = y1 + (rh * l8) // _OUT
            ye = y1 + (rh * (l8 + 1)) // _OUT
            ly = ye - yb
            kyv = ((ly >= 2).astype(jnp.int32) + (ly >= 4).astype(jnp.int32)
                   + (ly >= 8).astype(jnp.int32))
            yhi = ye - (one16 << kyv)
            yp_v[pl.ds(8, _LANES)] = kyv * plane + yb * w
            yp_v[pl.ds(8 + _LANES, _LANES)] = kyv * plane + yhi * w

            copies = []
            for by in range(_OUT):
                for sy in range(2):
                    yterm = plsc.load_gather(yp_v, [zero16 + (8 + sy * _LANES + by)])
                    idx = xbase + yterm
                    row = (by * 2 + sy) * _LANES
                    copies.append(pltpu.async_copy(
                        pyr_hbm.at[idx], buf_v.at[pl.ds(row, _LANES)], sem))
            for cp in copies:
                cp.wait()

            def cbody(cc, _):
                cs = cc * _LANES
                for by in range(_OUT):
                    rbase = (by * 2) * _LANES
                    for bx in range(_OUT):
                        v00 = buf_v[rbase + bx, pl.ds(cs, _LANES)]
                        v01 = buf_v[rbase + bx + 8, pl.ds(cs, _LANES)]
                        v10 = buf_v[rbase + _LANES + bx, pl.ds(cs, _LANES)]
                        v11 = buf_v[rbase + _LANES + bx + 8, pl.ds(cs, _LANES)]
                        acc_v[by * _OUT + bx, pl.ds(cs, _LANES)] = jnp.maximum(
                            jnp.maximum(v00, v01), jnp.maximum(v10, v11))
                return 0

            lax.fori_loop(0, c // _LANES, cbody, 0)

            gr = start + r

            @pl.when(gr < n_rois)
            def _():
                pltpu.sync_copy(acc_v, out_hbm.at[pl.ds(gr * _OROWS, _OROWS)])

            return carry

        lax.fori_loop(0, rpw, do_roi, 0)

    mesh = plsc.VectorSubcoreMesh(core_axis_name="c", subcore_axis_name="s",
                                  num_cores=_NC, num_subcores=_NS)
    return pl.kernel(
        body,
        out_type=jax.ShapeDtypeStruct((n_rois * _OROWS, c), jnp.float32),
        mesh=mesh,
        scratch_types=[
            pltpu.VMEM((rpw * 8,), jnp.int32),
            pltpu.VMEM((8 + 2 * _LANES,), jnp.int32),
            pltpu.VMEM((2 * _OUT * _LANES, c), jnp.float32),
            pltpu.VMEM((_OROWS, c), jnp.float32),
            pltpu.SemaphoreType.DMA,
        ],
        compiler_params=pltpu.CompilerParams(needs_layout_passes=False),
        interpret=interpret,
    )


def kernel(features, rois):
    _, c, h, w = features.shape
    n_rois = rois.shape[0]
    rpw = -(-n_rois // _NW)

    ft = jnp.transpose(features[0], (1, 2, 0))
    pyr = _build_pyramid(ft)
    pyr_rows = pyr.reshape(_NK * _NK * h * w, c)

    pad = jnp.tile(jnp.array([0, 0, 6, 6, 0, 0, 0, 0], jnp.int32), (rpw * _NW, 1))
    rois_p = pad.at[:n_rois, :4].set(rois.astype(jnp.int32)).reshape(-1)

    out_flat = _make_sc_pool(h, w, c, n_rois, rpw)(pyr_rows, rois_p)
    out = out_flat.reshape(n_rois, _OROWS, c)[:, :_OUT * _OUT]
    return out.reshape(n_rois, _OUT, _OUT, c).transpose(0, 3, 1, 2)


# speedup vs baseline: 42.0379x; 1.2908x over previous
import jax
import jax.numpy as jnp
from jax import lax
from jax.experimental import pallas as pl
from jax.experimental.pallas import tpu as pltpu
from jax.experimental.pallas import tpu_sc as plsc

_OUT = 7
_NK = 4
_NC = 2
_NS = 16
_NW = _NC * _NS
_LANES = 16


def _make_pyramid_body(h, w, wp, c):
    def body(ft_ref, out_ref, xt_ref, yt_ref):
        t = pl.program_id(0)
        kx = t // _NK
        ky = t % _NK

        @pl.when(t == 0)
        def _():
            xt_ref[...] = ft_ref[...]

        for k in range(1, _NK):
            s = 1 << (k - 1)

            @pl.when((ky == 0) & (kx == k))
            def _():
                xt = xt_ref[...]
                shifted = jnp.concatenate([xt[:, s:, :]] + [xt[:, -1:, :]] * s,
                                          axis=1)
                xt_ref[...] = jnp.maximum(xt, shifted)

        @pl.when(ky == 0)
        def _():
            xt = xt_ref[...]
            pad = jnp.full((h, wp - w, c), -jnp.inf, jnp.float32)
            yt_ref[...] = jnp.concatenate([xt, pad], axis=1).reshape(h * wp, c)

        for k in range(1, _NK):
            s = 1 << (k - 1)

            @pl.when(ky == k)
            def _():
                yt = yt_ref[...]
                shifted = jnp.concatenate([yt[s * wp:]] + [yt[-wp:]] * s, axis=0)
                yt_ref[...] = jnp.maximum(yt, shifted)

        out_ref[0] = yt_ref[...]

    return body


def _build_pyramid(ft, wp, interpret=False):
    h, w, c = ft.shape
    return pl.pallas_call(
        _make_pyramid_body(h, w, wp, c),
        grid=(_NK * _NK,),
        in_specs=[pl.BlockSpec((h, w, c), lambda t: (0, 0, 0))],
        out_specs=pl.BlockSpec((1, h * wp, c), lambda t: (t, 0, 0)),
        out_shape=jax.ShapeDtypeStruct((_NK * _NK, h * wp, c), jnp.float32),
        scratch_shapes=[
            pltpu.VMEM((h, w, c), jnp.float32),
            pltpu.VMEM((h * wp, c), jnp.float32),
        ],
        interpret=interpret,
    )(ft)


_OROWS = 56


def _make_sc_pool(h, wp, c, n_rois, rpw, interpret=False):
    plane = h * wp

    def body(pyr_hbm, rois_hbm, out_hbm, rois_v, yp_v, buf_v, acc_v, sem):
        cid = lax.axis_index("c")
        sid = lax.axis_index("s")
        wid = cid * _NS + sid
        start = wid * rpw

        pltpu.sync_copy(rois_hbm.at[pl.ds(start * 8, rpw * 8)], rois_v)

        lane = lax.iota(jnp.int32, _LANES)
        l8 = lane & 7
        hi_half = lane >= 8
        zero16 = jnp.zeros((_LANES,), jnp.int32)
        one16 = zero16 + 1

        def do_roi(r, carry):
            rv = zero16 + r * 8
            x1 = plsc.load_gather(rois_v, [rv])
            y1 = plsc.load_gather(rois_v, [rv + 1])
            x2 = plsc.load_gather(rois_v, [rv + 2])
            y2 = plsc.load_gather(rois_v, [rv + 3])
            rw = x2 - x1 + 1
            rh = y2 - y1 + 1

            xb = x1 + (rw * l8) // _OUT
            xe = x1 + (rw * (l8 + 1)) // _OUT
            lx = xe - xb
            kxv = ((lx >= 2).astype(jnp.int32) + (lx >= 4).astype(jnp.int32)
                   + (lx >= 8).astype(jnp.int32))
            xhi = xe - (one16 << kxv)
            xpt = jnp.where(hi_half, xhi, xb)
            xbase = kxv * (_NK * plane) + xpt

            yb = y1 + (rh * l8) // _OUT
            ye = y1 + (rh * (l8 + 1)) // _OUT
            ly = ye - yb
            kyv = ((ly >= 2).astype(jnp.int32) + (ly >= 4).astype(jnp.int32)
                   + (ly >= 8).astype(jnp.int32))
            yhi = ye - (one16 << kyv)
            yp_v[pl.ds(8, _LANES)] = kyv * plane + yb * wp
            yp_v[pl.ds(8 + _LANES, _LANES)] = kyv * plane + yhi * wp

            copies = []
            for by in range(_OUT):
                for sy in range(2):
                    yterm = plsc.load_gather(yp_v, [zero16 + (8 + sy * _LANES + by)])
                    idx = xbase + yterm
                    row = (by * 2 + sy) * _LANES
                    copies.append(pltpu.async_copy(
                        pyr_hbm.at[idx], buf_v.at[pl.ds(row, _LANES)], sem))
            for cp in copies:
                cp.wait()

            def cbody(cc, _):
                cs = cc * _LANES
                for by in range(_OUT):
                    rbase = (by * 2) * _LANES
                    for bx in range(_OUT):
                        v00 = buf_v[rbase + bx, pl.ds(cs, _LANES)]
                        v01 = buf_v[rbase + bx + 8, pl.ds(cs, _LANES)]
                        v10 = buf_v[rbase + _LANES + bx, pl.ds(cs, _LANES)]
                        v11 = buf_v[rbase + _LANES + bx + 8, pl.ds(cs, _LANES)]
                        acc_v[by * _OUT + bx, pl.ds(cs, _LANES)] = jnp.maximum(
                            jnp.maximum(v00, v01), jnp.maximum(v10, v11))
                return 0

            lax.fori_loop(0, c // _LANES, cbody, 0)

            gr = start + r

            @pl.when(gr < n_rois)
            def _():
                pltpu.sync_copy(acc_v, out_hbm.at[pl.ds(gr * _OROWS, _OROWS)])

            return carry

        lax.fori_loop(0, rpw, do_roi, 0)

    mesh = plsc.VectorSubcoreMesh(core_axis_name="c", subcore_axis_name="s",
                                  num_cores=_NC, num_subcores=_NS)
    return pl.kernel(
        body,
        out_type=jax.ShapeDtypeStruct((n_rois * _OROWS, c), jnp.float32),
        mesh=mesh,
        scratch_types=[
            pltpu.VMEM((rpw * 8,), jnp.int32),
            pltpu.VMEM((8 + 2 * _LANES,), jnp.int32),
            pltpu.VMEM((2 * _OUT * _LANES, c), jnp.float32),
            pltpu.VMEM((_OROWS, c), jnp.float32),
            pltpu.SemaphoreType.DMA,
        ],
        compiler_params=pltpu.CompilerParams(needs_layout_passes=False),
        interpret=interpret,
    )


def kernel(features, rois):
    _, c, h, w = features.shape
    n_rois = rois.shape[0]
    rpw = -(-n_rois // _NW)

    wp = -(-w // 8) * 8
    ft = jnp.transpose(features[0], (1, 2, 0))
    pyr = _build_pyramid(ft, wp)
    pyr_rows = pyr.reshape(_NK * _NK * h * wp, c)

    pad = jnp.tile(jnp.array([0, 0, 6, 6, 0, 0, 0, 0], jnp.int32), (rpw * _NW, 1))
    rois_p = pad.at[:n_rois, :4].set(rois.astype(jnp.int32)).reshape(-1)

    out_flat = _make_sc_pool(h, wp, c, n_rois, rpw)(pyr_rows, rois_p)
    out = out_flat.reshape(n_rois, _OROWS, c)[:, :_OUT * _OUT]
    return out.reshape(n_rois, _OUT, _OUT, c).transpose(0, 3, 1, 2)


# speedup vs baseline: 51.0798x; 1.2151x over previous
import jax
import jax.numpy as jnp
from jax import lax
from jax.experimental import pallas as pl
from jax.experimental.pallas import tpu as pltpu
from jax.experimental.pallas import tpu_sc as plsc

_OUT = 7
_NK = 4
_NC = 2
_NS = 16
_NW = _NC * _NS
_LANES = 16


def _make_pyramid_body(h, w, wp, c):
    def body(ft_ref, out_ref, xt_ref, yt_ref):
        t = pl.program_id(0)
        kx = t // _NK
        ky = t % _NK

        @pl.when(t == 0)
        def _():
            xt_ref[...] = ft_ref[...]

        for k in range(1, _NK):
            s = 1 << (k - 1)

            @pl.when((ky == 0) & (kx == k))
            def _():
                xt = xt_ref[...]
                shifted = jnp.concatenate([xt[:, s:, :]] + [xt[:, -1:, :]] * s,
                                          axis=1)
                xt_ref[...] = jnp.maximum(xt, shifted)

        @pl.when(ky == 0)
        def _():
            xt = xt_ref[...]
            pad = jnp.full((h, wp - w, c), -jnp.inf, jnp.float32)
            yt_ref[...] = jnp.concatenate([xt, pad], axis=1).reshape(h * wp, c)

        for k in range(1, _NK):
            s = 1 << (k - 1)

            @pl.when(ky == k)
            def _():
                yt = yt_ref[...]
                shifted = jnp.concatenate([yt[s * wp:]] + [yt[-wp:]] * s, axis=0)
                yt_ref[...] = jnp.maximum(yt, shifted)

        out_ref[0] = yt_ref[...]

    return body


def _build_pyramid(ft, wp, interpret=False):
    h, w, c = ft.shape
    return pl.pallas_call(
        _make_pyramid_body(h, w, wp, c),
        grid=(_NK * _NK,),
        in_specs=[pl.BlockSpec((h, w, c), lambda t: (0, 0, 0))],
        out_specs=pl.BlockSpec((1, h * wp, c), lambda t: (t, 0, 0)),
        out_shape=jax.ShapeDtypeStruct((_NK * _NK, h * wp, c), jnp.float32),
        scratch_shapes=[
            pltpu.VMEM((h, w, c), jnp.float32),
            pltpu.VMEM((h * wp, c), jnp.float32),
        ],
        interpret=interpret,
    )(ft)


_OROWS = 56


def _make_sc_pool(h, wp, c, n_rois, rpw, interpret=False):
    plane = h * wp

    def body(pyr_hbm, rois_hbm, out_hbm, rois_v, yp_v, buf0_v, buf1_v, acc_v,
             sem0, sem1):
        cid = lax.axis_index("c")
        sid = lax.axis_index("s")
        wid = cid * _NS + sid
        start = wid * rpw

        pltpu.sync_copy(rois_hbm.at[pl.ds(start * 8, rpw * 8)],
                        rois_v.at[pl.ds(8, rpw * 8)])

        lane = lax.iota(jnp.int32, _LANES)
        l8 = lane & 7
        hi_half = lane >= 8
        zero16 = jnp.zeros((_LANES,), jnp.int32)
        one16 = zero16 + 1

        def issue(r, buf_v, sem):
            rv = zero16 + (8 + r * 8)
            x1 = plsc.load_gather(rois_v, [rv])
            y1 = plsc.load_gather(rois_v, [rv + 1])
            x2 = plsc.load_gather(rois_v, [rv + 2])
            y2 = plsc.load_gather(rois_v, [rv + 3])
            rw = x2 - x1 + 1
            rh = y2 - y1 + 1

            xb = x1 + (rw * l8) // _OUT
            xe = x1 + (rw * (l8 + 1)) // _OUT
            lx = xe - xb
            kxv = ((lx >= 2).astype(jnp.int32) + (lx >= 4).astype(jnp.int32)
                   + (lx >= 8).astype(jnp.int32))
            xhi = xe - (one16 << kxv)
            xpt = jnp.where(hi_half, xhi, xb)
            xbase = kxv * (_NK * plane) + xpt

            yb = y1 + (rh * l8) // _OUT
            ye = y1 + (rh * (l8 + 1)) // _OUT
            ly = ye - yb
            kyv = ((ly >= 2).astype(jnp.int32) + (ly >= 4).astype(jnp.int32)
                   + (ly >= 8).astype(jnp.int32))
            yhi = ye - (one16 << kyv)
            yp_v[pl.ds(8, _LANES)] = kyv * plane + yb * wp
            yp_v[pl.ds(8 + _LANES, _LANES)] = kyv * plane + yhi * wp

            for by in range(_OUT):
                for sy in range(2):
                    yterm = plsc.load_gather(yp_v, [zero16 + (8 + sy * _LANES + by)])
                    idx = xbase + yterm
                    row = (by * 2 + sy) * _LANES
                    pltpu.async_copy(pyr_hbm.at[idx],
                                     buf_v.at[pl.ds(row, _LANES)], sem)

        def wait_set(buf_v, sem):
            for j in range(2 * _OUT):
                pltpu.make_async_copy(
                    pyr_hbm.at[pl.ds(0, _LANES)],
                    buf_v.at[pl.ds(j * _LANES, _LANES)], sem).wait()

        def compute_out(r, buf_v):
            def cbody(cc, _):
                cs = cc * _LANES
                for by in range(_OUT):
                    rbase = (by * 2) * _LANES
                    for bx in range(_OUT):
                        v00 = buf_v[rbase + bx, pl.ds(cs, _LANES)]
                        v01 = buf_v[rbase + bx + 8, pl.ds(cs, _LANES)]
                        v10 = buf_v[rbase + _LANES + bx, pl.ds(cs, _LANES)]
                        v11 = buf_v[rbase + _LANES + bx + 8, pl.ds(cs, _LANES)]
                        acc_v[by * _OUT + bx, pl.ds(cs, _LANES)] = jnp.maximum(
                            jnp.maximum(v00, v01), jnp.maximum(v10, v11))
                return 0

            lax.fori_loop(0, c // _LANES, cbody, 0)

            gr = start + r

            @pl.when(gr < n_rois)
            def _():
                pltpu.sync_copy(acc_v, out_hbm.at[pl.ds(gr * _OROWS, _OROWS)])

        last = rpw - 1

        def pair_body(p, carry):
            r0 = p * 2
            r1 = r0 + 1
            issue(r1, buf1_v, sem1)
            wait_set(buf0_v, sem0)
            compute_out(r0, buf0_v)
            issue(jnp.minimum(r0 + 2, last), buf0_v, sem0)
            wait_set(buf1_v, sem1)
            compute_out(r1, buf1_v)
            return carry

        issue(0, buf0_v, sem0)
        lax.fori_loop(0, rpw // 2, pair_body, 0)
        wait_set(buf0_v, sem0)

    mesh = plsc.VectorSubcoreMesh(core_axis_name="c", subcore_axis_name="s",
                                  num_cores=_NC, num_subcores=_NS)
    return pl.kernel(
        body,
        out_type=jax.ShapeDtypeStruct((n_rois * _OROWS, c), jnp.float32),
        mesh=mesh,
        scratch_types=[
            pltpu.VMEM((8 + rpw * 8,), jnp.int32),
            pltpu.VMEM((8 + 2 * _LANES,), jnp.int32),
            pltpu.VMEM((2 * _OUT * _LANES, c), jnp.float32),
            pltpu.VMEM((2 * _OUT * _LANES, c), jnp.float32),
            pltpu.VMEM((_OROWS, c), jnp.float32),
            pltpu.SemaphoreType.DMA,
            pltpu.SemaphoreType.DMA,
        ],
        compiler_params=pltpu.CompilerParams(needs_layout_passes=False),
        interpret=interpret,
    )


def kernel(features, rois):
    _, c, h, w = features.shape
    n_rois = rois.shape[0]
    rpw = -(-n_rois // _NW)
    rpw += rpw & 1

    wp = -(-w // 8) * 8
    ft = jnp.transpose(features[0], (1, 2, 0))
    pyr = _build_pyramid(ft, wp)
    pyr_rows = pyr.reshape(_NK * _NK * h * wp, c)

    pad = jnp.tile(jnp.array([0, 0, 6, 6, 0, 0, 0, 0], jnp.int32), (rpw * _NW, 1))
    rois_p = pad.at[:n_rois, :4].set(rois.astype(jnp.int32)).reshape(-1)

    out_flat = _make_sc_pool(h, wp, c, n_rois, rpw)(pyr_rows, rois_p)
    out = out_flat.reshape(n_rois, _OROWS, c)[:, :_OUT * _OUT]
    return out.reshape(n_rois, _OUT, _OUT, c).transpose(0, 3, 1, 2)
